# Initial kernel scaffold; baseline (speedup 1.0000x reference)
import jax
import jax.numpy as jnp
from jax.experimental import pallas as pl


def kernel(user_nodes, pos_item_nodes, neg_item_nodes, edge_index, user_index_5, v_feat, a_feat, t_feat, v_preference, a_preference, t_preference, W_mlp_v, b_mlp_v, W_mlp_a, b_mlp_a, W_mlp_t, b_mlp_t, W_conv_v, W_conv_a, W_conv_t, W_ug):
    raise NotImplementedError("write your pallas kernel here")


# speedup vs baseline: 33.3646x; 33.3646x over previous
import functools

import jax
import jax.numpy as jnp
from jax import lax
from jax.experimental import pallas as pl
from jax.experimental.pallas import tpu as pltpu
from jax.experimental.pallas import tpu_sc as plsc

NUSR = 2000
NITM = 8000
NN = NUSR + NITM
D = 64
DF = 128
EU = 10000
BTR = 4096

NC = 2
NS = 16
NW = NC * NS

CH = 128
E2 = 2 * 640000
CPT = -(-E2 // (NW * CH))
EPT = CPT * CH
E2P = EPT * NW
HROWS = NN + 112
RPT_INIT = HROWS // NS
RPT_OUT = 624

UCH = 125
UCPT = EU // (NS * UCH)
URPT = 128
ULAST = NUSR - 15 * URPT
IRPT = 512
ILAST = NITM - 15 * IRPT

_mesh = plsc.VectorSubcoreMesh(core_axis_name="c", subcore_axis_name="s")
_SC_PARAMS = pltpu.CompilerParams(use_tc_tiling_on_sc=False, needs_layout_passes=False)


def _l2n(x):
    n = jnp.sqrt(jnp.sum(x * x, axis=1, keepdims=True))
    return x / jnp.maximum(n, 1e-12)


def _users_body(pv, pa, pt_, cv, ca, ct, x_out, xw_out):
    xv = _l2n(pv[...])
    xa = _l2n(pa[...])
    xt = _l2n(pt_[...])
    x_out[...] = xv + xa + xt
    xw_out[...] = (
        jnp.dot(xv, cv[...], preferred_element_type=jnp.float32)
        + jnp.dot(xa, ca[...], preferred_element_type=jnp.float32)
        + jnp.dot(xt, ct[...], preferred_element_type=jnp.float32))


def _items_body(fv, fa, ft, wv, wa, wt, bv, ba, bt, cv, ca, ct, x_out, xw_out):
    def branch(f, w, b, c):
        t = jnp.dot(f[...], w[...], preferred_element_type=jnp.float32) + b[...]
        x = _l2n(t)
        return x, jnp.dot(x, c[...], preferred_element_type=jnp.float32)

    xv, xwv = branch(fv, wv, bv, cv)
    xa, xwa = branch(fa, wa, ba, ca)
    xt, xwt = branch(ft, wt, bt, ct)
    x_out[...] = xv + xa + xt
    xw_out[...] = xwv + xwa + xwt


@functools.partial(
    pl.kernel,
    out_type=jax.ShapeDtypeStruct((NC * NN, D), jnp.float32),
    mesh=_mesh,
    compiler_params=_SC_PARAMS,
    scratch_types=[
        pltpu.VMEM((CPT, CH), jnp.int32),
        pltpu.VMEM((CPT, CH), jnp.int32),
        pltpu.VMEM((CH, D), jnp.float32),
        pltpu.VMEM_SHARED((HROWS, D), jnp.float32),
        pltpu.SemaphoreType.DMA,
    ],
)
def _edge_scatter(xw_hbm, xinit_hbm, zeros_hbm, src_hbm, dst_hbm, out_hbm,
                  srcv, dstv, rows, hsp, sem):
    c = lax.axis_index("c")
    s = lax.axis_index("s")
    wid = c * NS + s
    r0 = s * RPT_INIT

    @pl.when(c == 0)
    def _():
        pltpu.sync_copy(xinit_hbm.at[pl.ds(r0, RPT_INIT)],
                        hsp.at[pl.ds(r0, RPT_INIT)])

    @pl.when(c != 0)
    def _():
        pltpu.sync_copy(zeros_hbm.at[pl.ds(r0, RPT_INIT)],
                        hsp.at[pl.ds(r0, RPT_INIT)])

    pltpu.sync_copy(src_hbm.at[wid], srcv)
    pltpu.sync_copy(dst_hbm.at[wid], dstv)
    plsc.subcore_barrier()

    def body(j, carry):
        pltpu.async_copy(xw_hbm.at[srcv.at[j]], rows, sem).wait()
        pltpu.sync_copy(rows, hsp.at[dstv.at[j]], add=True)
        return carry

    lax.fori_loop(0, CPT, body, 0)
    plsc.subcore_barrier()

    o0 = s * RPT_OUT
    pltpu.sync_copy(hsp.at[pl.ds(o0, RPT_OUT)],
                    out_hbm.at[pl.ds(c * NN + o0, RPT_OUT)])

    @pl.when(s == NS - 1)
    def _():
        tail = NS * RPT_OUT
        pltpu.sync_copy(hsp.at[pl.ds(tail, NN - tail)],
                        out_hbm.at[pl.ds(c * NN + tail, NN - tail)])


def _combine_body(h0, h1, wug, rep_out, y1_out, y2_out):
    rep = h0[...] + h1[...]
    rep_out[...] = rep
    u = rep[0:NUSR]
    y1 = jnp.dot(u, wug[...], preferred_element_type=jnp.float32)
    y1_out[...] = y1
    y2_out[...] = jnp.dot(y1, wug[...], preferred_element_type=jnp.float32)


@functools.partial(
    pl.kernel,
    out_type=jax.ShapeDtypeStruct((NUSR, D), jnp.float32),
    mesh=_mesh,
    compiler_params=_SC_PARAMS,
    scratch_types=[
        pltpu.VMEM((UCPT, UCH), jnp.int32),
        pltpu.VMEM((UCPT, UCH), jnp.int32),
        pltpu.VMEM((UCH, D), jnp.float32),
        pltpu.VMEM_SHARED((NUSR, D), jnp.float32),
        pltpu.SemaphoreType.DMA,
    ],
)
def _ug_first(y2_hbm, zeros_hbm, usrc_hbm, udst_hbm, t_out,
              srcv, dstv, rows, tsp, sem):
    c = lax.axis_index("c")
    s = lax.axis_index("s")

    @pl.when(c == 0)
    def _():
        r0 = s * URPT

        @pl.when(s < NS - 1)
        def _():
            pltpu.sync_copy(zeros_hbm.at[pl.ds(r0, URPT)],
                            tsp.at[pl.ds(r0, URPT)])

        @pl.when(s == NS - 1)
        def _():
            pltpu.sync_copy(zeros_hbm.at[pl.ds(r0, ULAST)],
                            tsp.at[pl.ds(r0, ULAST)])

        pltpu.sync_copy(usrc_hbm.at[s], srcv)
        pltpu.sync_copy(udst_hbm.at[s], dstv)
        plsc.subcore_barrier()

        def body(j, carry):
            pltpu.async_copy(y2_hbm.at[srcv.at[j]], rows, sem).wait()
            pltpu.sync_copy(rows, tsp.at[dstv.at[j]], add=True)
            return carry

        lax.fori_loop(0, UCPT, body, 0)
        plsc.subcore_barrier()

        @pl.when(s < NS - 1)
        def _():
            pltpu.sync_copy(tsp.at[pl.ds(r0, URPT)], t_out.at[pl.ds(r0, URPT)])

        @pl.when(s == NS - 1)
        def _():
            pltpu.sync_copy(tsp.at[pl.ds(r0, ULAST)], t_out.at[pl.ds(r0, ULAST)])


@functools.partial(
    pl.kernel,
    out_type=jax.ShapeDtypeStruct((NN, D), jnp.float32),
    mesh=_mesh,
    compiler_params=_SC_PARAMS,
    scratch_types=[
        pltpu.VMEM((UCPT, UCH), jnp.int32),
        pltpu.VMEM((UCPT, UCH), jnp.int32),
        pltpu.VMEM((UCH, D), jnp.float32),
        pltpu.VMEM_SHARED((NUSR, D), jnp.float32),
        pltpu.VMEM((IRPT, D), jnp.float32),
        pltpu.SemaphoreType.DMA,
    ],
)
def _ug_second(y1_hbm, t_hbm, rep_hbm, usrc_hbm, udst_hbm, res_out,
               srcv, dstv, rows, accsp, cpbuf, sem):
    c = lax.axis_index("c")
    s = lax.axis_index("s")

    @pl.when(c == 0)
    def _():
        r0 = s * URPT

        @pl.when(s < NS - 1)
        def _():
            pltpu.sync_copy(rep_hbm.at[pl.ds(r0, URPT)],
                            accsp.at[pl.ds(r0, URPT)])

        @pl.when(s == NS - 1)
        def _():
            pltpu.sync_copy(rep_hbm.at[pl.ds(r0, ULAST)],
                            accsp.at[pl.ds(r0, ULAST)])

        pltpu.sync_copy(usrc_hbm.at[s], srcv)
        pltpu.sync_copy(udst_hbm.at[s], dstv)
        plsc.subcore_barrier()

        def body(j, carry):
            pltpu.async_copy(y1_hbm.at[srcv.at[j]], rows, sem).wait()
            pltpu.sync_copy(rows, accsp.at[dstv.at[j]], add=True)
            pltpu.async_copy(t_hbm.at[srcv.at[j]], rows, sem).wait()
            pltpu.sync_copy(rows, accsp.at[dstv.at[j]], add=True)
            return carry

        lax.fori_loop(0, UCPT, body, 0)
        plsc.subcore_barrier()

        @pl.when(s < NS - 1)
        def _():
            pltpu.sync_copy(accsp.at[pl.ds(r0, URPT)],
                            res_out.at[pl.ds(r0, URPT)])

        @pl.when(s == NS - 1)
        def _():
            pltpu.sync_copy(accsp.at[pl.ds(r0, ULAST)],
                            res_out.at[pl.ds(r0, ULAST)])

    @pl.when(c != 0)
    def _():
        b = NUSR + s * IRPT

        @pl.when(s < NS - 1)
        def _():
            pltpu.sync_copy(rep_hbm.at[pl.ds(b, IRPT)], cpbuf)
            pltpu.sync_copy(cpbuf, res_out.at[pl.ds(b, IRPT)])

        @pl.when(s == NS - 1)
        def _():
            pltpu.sync_copy(rep_hbm.at[pl.ds(b, ILAST)], cpbuf.at[pl.ds(0, ILAST)])
            pltpu.sync_copy(cpbuf.at[pl.ds(0, ILAST)], res_out.at[pl.ds(b, ILAST)])


TPT = BTR // NW


@functools.partial(
    pl.kernel,
    out_type=(jax.ShapeDtypeStruct((BTR,), jnp.float32),
              jax.ShapeDtypeStruct((BTR,), jnp.float32)),
    mesh=_mesh,
    compiler_params=_SC_PARAMS,
    scratch_types=[
        pltpu.VMEM((1, TPT), jnp.int32),
        pltpu.VMEM((1, TPT), jnp.int32),
        pltpu.VMEM((1, TPT), jnp.int32),
        pltpu.VMEM((TPT, D), jnp.float32),
        pltpu.VMEM((TPT, D), jnp.float32),
        pltpu.VMEM((TPT, D), jnp.float32),
        pltpu.VMEM((TPT,), jnp.float32),
        pltpu.VMEM((TPT,), jnp.float32),
        pltpu.SemaphoreType.DMA,
    ],
)
def _triplet_dots(res_hbm, u3, p3, n3, pos_out, neg_out,
                  uidx, pidx, nidx, urows, prows, nrows, posv, negv, sem):
    c = lax.axis_index("c")
    s = lax.axis_index("s")
    wid = c * NS + s
    pltpu.sync_copy(u3.at[wid], uidx)
    pltpu.sync_copy(p3.at[wid], pidx)
    pltpu.sync_copy(n3.at[wid], nidx)
    pltpu.async_copy(res_hbm.at[uidx.at[0]], urows, sem).wait()
    pltpu.async_copy(res_hbm.at[pidx.at[0]], prows, sem).wait()
    pltpu.async_copy(res_hbm.at[nidx.at[0]], nrows, sem).wait()

    def group(g, carry):
        rvec = lax.iota(jnp.int32, 16) + g * 16

        def col(k, accs):
            accp, accn = accs
            cvec = jnp.full((16,), 0, jnp.int32) + k
            uv = plsc.load_gather(urows, [rvec, cvec])
            pv = plsc.load_gather(prows, [rvec, cvec])
            nv = plsc.load_gather(nrows, [rvec, cvec])
            return accp + uv * pv, accn + uv * nv

        accp, accn = lax.fori_loop(
            0, D, col,
            (jnp.zeros((16,), jnp.float32), jnp.zeros((16,), jnp.float32)))
        posv[pl.ds(g * 16, 16)] = accp
        negv[pl.ds(g * 16, 16)] = accn
        return carry

    lax.fori_loop(0, TPT // 16, group, 0)
    pltpu.sync_copy(posv, pos_out.at[pl.ds(wid * TPT, TPT)])
    pltpu.sync_copy(negv, neg_out.at[pl.ds(wid * TPT, TPT)])


def kernel(user_nodes, pos_item_nodes, neg_item_nodes, edge_index,
           user_index_5, v_feat, a_feat, t_feat, v_preference, a_preference,
           t_preference, W_mlp_v, b_mlp_v, W_mlp_a, b_mlp_a, W_mlp_t, b_mlp_t,
           W_conv_v, W_conv_a, W_conv_t, W_ug):
    f32 = jnp.float32

    x_u, xw_u = pl.pallas_call(
        _users_body,
        out_shape=(jax.ShapeDtypeStruct((NUSR, D), f32),
                   jax.ShapeDtypeStruct((NUSR, D), f32)),
    )(v_preference, a_preference, t_preference, W_conv_v, W_conv_a, W_conv_t)

    x_i, xw_i = pl.pallas_call(
        _items_body,
        out_shape=(jax.ShapeDtypeStruct((NITM, D), f32),
                   jax.ShapeDtypeStruct((NITM, D), f32)),
    )(v_feat, a_feat, t_feat, W_mlp_v, W_mlp_a, W_mlp_t,
      b_mlp_v.reshape(1, D), b_mlp_a.reshape(1, D), b_mlp_t.reshape(1, D),
      W_conv_v, W_conv_a, W_conv_t)

    xw = jnp.concatenate([xw_u, xw_i], axis=0)
    x_init = jnp.concatenate([x_u, x_i, jnp.zeros((HROWS - NN, D), f32)], axis=0)
    zeros_big = jnp.zeros((HROWS, D), f32)

    ei = edge_index.astype(jnp.int32)
    pad = E2P - E2
    src3 = jnp.concatenate(
        [ei[0], ei[1], jnp.zeros((pad,), jnp.int32)]).reshape(NW, CPT, CH)
    dst3 = jnp.concatenate(
        [ei[1], ei[0], jnp.full((pad,), NN, jnp.int32)]).reshape(NW, CPT, CH)

    hflat = _edge_scatter(xw, x_init, zeros_big, src3, dst3)
    rep, y1, y2 = pl.pallas_call(
        _combine_body,
        out_shape=(jax.ShapeDtypeStruct((NN, D), f32),
                   jax.ShapeDtypeStruct((NUSR, D), f32),
                   jax.ShapeDtypeStruct((NUSR, D), f32)),
    )(hflat[:NN], hflat[NN:], W_ug)

    ui = user_index_5.astype(jnp.int32)
    usrc3 = ui[0].reshape(NS, UCPT, UCH)
    udst3 = ui[1].reshape(NS, UCPT, UCH)

    t_arr = _ug_first(y2, zeros_big[:NUSR], usrc3, udst3)
    result = _ug_second(y1, t_arr, rep, usrc3, udst3)

    u3 = user_nodes.astype(jnp.int32).reshape(NW, 1, TPT)
    p3 = pos_item_nodes.astype(jnp.int32).reshape(NW, 1, TPT)
    n3 = neg_item_nodes.astype(jnp.int32).reshape(NW, 1, TPT)
    pos, neg = _triplet_dots(result, u3, p3, n3)
    return pos, neg
